# baseline (speedup 1.0000x reference)
import functools

import jax
import jax.numpy as jnp
from jax import lax
from jax.experimental import pallas as pl
from jax.experimental.pallas import tpu as pltpu
from jax.experimental.pallas import tpu_sc as plsc

DIM = 64
PDIM = 128
SCALE = 8.0

try:
    _info = plsc.get_sparse_core_info()
    _NC = _info.num_cores
    _NS = _info.num_subcores
    _L = _info.num_lanes
except ValueError:
    _NC, _NS, _L = 2, 16, 16
_NW = _NC * _NS
LB = 128


@functools.lru_cache(maxsize=None)
def _make_gather(T: int, NB: int, V: int):
    assert NB == _NW

    mesh = plsc.VectorSubcoreMesh(
        core_axis_name="c", subcore_axis_name="s", num_cores=_NC
    )

    @functools.partial(
        pl.kernel,
        mesh=mesh,
        compiler_params=pltpu.CompilerParams(
            use_tc_tiling_on_sc=True, needs_layout_passes=False
        ),
        out_type=jax.ShapeDtypeStruct((T, DIM, NB * LB), jnp.float32),
        scratch_types=[
            pltpu.VMEM((T, LB), jnp.int32),
            pltpu.VMEM((LB, PDIM), jnp.float32),
            pltpu.VMEM((LB, PDIM), jnp.float32),
            pltpu.VMEM((DIM, LB), jnp.float32),
            pltpu.VMEM((DIM, LB), jnp.float32),
            pltpu.SemaphoreType.DMA,
            pltpu.SemaphoreType.DMA,
            pltpu.SemaphoreType.DMA,
            pltpu.SemaphoreType.DMA,
        ],
    )
    def emb(xt_hbm, table_hbm, out_hbm, idx_v, in0, in1, out0, out1,
            gsem0, gsem1, ssem0, ssem1):
        ins = (in0, in1)
        outs = (out0, out1)
        gsems = (gsem0, gsem1)
        ssems = (ssem0, ssem1)

        wid = lax.axis_index("s") * _NC + lax.axis_index("c")
        lane0 = wid * LB
        pltpu.sync_copy(xt_hbm.at[:, pl.ds(lane0, LB)], idx_v)

        def gather_start(t, buf, sem):
            pltpu.async_copy(table_hbm.at[idx_v.at[t]], buf, sem)

        for b in range(2):
            gather_start(b, ins[b], gsems[b])

        iota = lax.iota(jnp.int32, _L)
        zeros = jnp.zeros((_L,), jnp.int32)

        def body(t0, carry):
            for b in range(2):
                t = t0 * 2 + b
                inb, outb = ins[b], outs[b]
                pltpu.make_async_copy(
                    table_hbm.at[idx_v.at[t]], inb, gsems[b]
                ).wait()

                @pl.when(t0 > 0)
                def _():
                    pltpu.make_async_copy(
                        outb,
                        out_hbm.at[t, :, pl.ds(lane0, LB)],
                        ssems[b],
                    ).wait()

                def _xp(d, c):
                    dvec = zeros + d
                    for b0 in range(LB // _L):
                        v = plsc.load_gather(inb, [iota + b0 * _L, dvec])
                        outb[d, pl.ds(b0 * _L, _L)] = v * SCALE
                    return c

                lax.fori_loop(0, DIM, _xp, 0)

                pltpu.async_copy(
                    outb, out_hbm.at[t, :, pl.ds(lane0, LB)], ssems[b]
                )

                @pl.when(t + 2 < T)
                def _():
                    gather_start(t + 2, inb, gsems[b])

            return carry

        lax.fori_loop(0, T // 2, body, 0)

        for b in range(2):
            pltpu.make_async_copy(
                outs[b], out_hbm.at[0, :, pl.ds(lane0, LB)], ssems[b]
            ).wait()

    return emb


@functools.lru_cache(maxsize=None)
def _make_transpose(V: int):
    blk = 8192
    grid = (V + blk - 1) // blk

    def tk(tab_ref, out_ref):
        t = jnp.transpose(tab_ref[...], (1, 0))
        out_ref[...] = jnp.concatenate(
            [t, jnp.zeros((blk, PDIM - DIM), jnp.float32)], axis=1
        )

    return pl.pallas_call(
        tk,
        grid=(grid,),
        in_specs=[pl.BlockSpec((DIM, blk), lambda i: (0, i))],
        out_specs=pl.BlockSpec((blk, PDIM), lambda i: (i, 0)),
        out_shape=jax.ShapeDtypeStruct((V, PDIM), jnp.float32),
    )


def kernel(x, table):
    NB, T = x.shape[0] // LB, x.shape[1]
    V = table.shape[0]
    tab2 = _make_transpose(V)(table.T)
    xt = x.T.astype(jnp.int32)
    p = _make_gather(T, NB, V)(xt, tab2)
    return jnp.transpose(p, (2, 0, 1))


# speedup vs baseline: 2.1923x; 2.1923x over previous
import functools

import jax
import jax.numpy as jnp
from jax import lax
from jax.experimental import pallas as pl
from jax.experimental.pallas import tpu as pltpu
from jax.experimental.pallas import tpu_sc as plsc

DIM = 64
PDIM = 128
SCALE = 8.0

try:
    _info = plsc.get_sparse_core_info()
    _NC = _info.num_cores
    _NS = _info.num_subcores
    _L = _info.num_lanes
except ValueError:
    _NC, _NS, _L = 2, 16, 16
_NW = _NC * _NS


@functools.lru_cache(maxsize=None)
def _make_kernel(B: int, V: int):
    assert B % _NW == 0
    bpw = B // _NW
    chunk = 200
    assert bpw % (2 * chunk) == 0
    nch = bpw // chunk
    unroll = 4

    mesh = plsc.VectorSubcoreMesh(
        core_axis_name="c", subcore_axis_name="s", num_cores=_NC
    )

    @functools.partial(
        pl.kernel,
        mesh=mesh,
        compiler_params=pltpu.CompilerParams(use_tc_tiling_on_sc=True),
        out_type=jax.ShapeDtypeStruct((B, DIM), jnp.float32),
        scratch_types=[
            pltpu.VMEM((bpw,), jnp.int32),
            pltpu.VMEM((chunk, PDIM), jnp.float32),
            pltpu.VMEM((chunk, PDIM), jnp.float32),
            pltpu.VMEM((chunk, DIM), jnp.float32),
            pltpu.VMEM((chunk, DIM), jnp.float32),
            pltpu.SemaphoreType.DMA,
            pltpu.SemaphoreType.DMA,
            pltpu.SemaphoreType.DMA,
            pltpu.SemaphoreType.DMA,
        ],
    )
    def emb(x_hbm, table_hbm, out_hbm, idx_v, in0, in1, out0, out1,
            gsem0, gsem1, ssem0, ssem1):
        ins = (in0, in1)
        outs = (out0, out1)
        gsems = (gsem0, gsem1)
        ssems = (ssem0, ssem1)

        wid = lax.axis_index("s") * _NC + lax.axis_index("c")
        base = wid * bpw
        pltpu.sync_copy(x_hbm.at[pl.ds(base, bpw)], idx_v)

        def gather_start(off, buf, sem):
            pltpu.async_copy(table_hbm.at[idx_v.at[pl.ds(off, chunk)]], buf, sem)

        for b in range(2):
            gather_start(pl.multiple_of(b * chunk, chunk), ins[b], gsems[b])

        def body(g0, carry):
            for b in range(2):
                g = g0 * 2 + b
                off = pl.multiple_of(g * chunk, chunk)
                inb, outb = ins[b], outs[b]
                pltpu.make_async_copy(
                    table_hbm.at[idx_v.at[pl.ds(off, chunk)]], inb, gsems[b]
                ).wait()

                @pl.when(g0 > 0)
                def _():
                    pltpu.make_async_copy(
                        outb, out_hbm.at[pl.ds(base, chunk)], ssems[b]
                    ).wait()

                def _scale(i0, c):
                    for u in range(unroll):
                        i = i0 * unroll + u
                        for j in range(DIM // _L):
                            sl = pl.ds(j * _L, _L)
                            outb[i, sl] = inb[i, sl] * SCALE
                    return c

                lax.fori_loop(0, chunk // unroll, _scale, 0)

                pltpu.async_copy(outb, out_hbm.at[pl.ds(base + off, chunk)], ssems[b])

                @pl.when(g + 2 < nch)
                def _():
                    off2 = pl.multiple_of((g + 2) * chunk, chunk)
                    gather_start(off2, inb, gsems[b])

            return carry

        lax.fori_loop(0, nch // 2, body, 0)

        for b in range(2):
            pltpu.make_async_copy(
                outs[b], out_hbm.at[pl.ds(base, chunk)], ssems[b]
            ).wait()

    return emb


@functools.lru_cache(maxsize=None)
def _make_transpose(V: int):
    blk = 16384
    grid = (V + blk - 1) // blk

    def tk(tab_ref, out_ref):
        t = jnp.transpose(tab_ref[...], (1, 0))
        out_ref[...] = jnp.concatenate(
            [t, jnp.zeros((blk, PDIM - DIM), jnp.float32)], axis=1
        )

    return pl.pallas_call(
        tk,
        grid=(grid,),
        in_specs=[pl.BlockSpec((DIM, blk), lambda i: (0, i))],
        out_specs=pl.BlockSpec((blk, PDIM), lambda i: (i, 0)),
        out_shape=jax.ShapeDtypeStruct((V, PDIM), jnp.float32),
    )


def kernel(x, table):
    B = x.size
    V = table.shape[0]
    flat = x.reshape(B).astype(jnp.int32)
    tab2 = _make_transpose(V)(table.T)
    out = _make_kernel(B, V)(flat, tab2)
    return out.reshape(x.shape + (DIM,))


# speedup vs baseline: 2.2106x; 1.0084x over previous
import functools

import jax
import jax.numpy as jnp
from jax import lax
from jax.experimental import pallas as pl
from jax.experimental.pallas import tpu as pltpu
from jax.experimental.pallas import tpu_sc as plsc

DIM = 64
PDIM = 128
SCALE = 8.0

try:
    _info = plsc.get_sparse_core_info()
    _NC = _info.num_cores
    _NS = _info.num_subcores
    _L = _info.num_lanes
except ValueError:
    _NC, _NS, _L = 2, 16, 16
_NW = _NC * _NS


@functools.lru_cache(maxsize=None)
def _make_kernel(B: int, V: int):
    assert B % _NW == 0
    bpw = B // _NW
    chunk = 200
    assert bpw % (2 * chunk) == 0
    nch = bpw // chunk
    unroll = 4

    mesh = plsc.VectorSubcoreMesh(
        core_axis_name="c", subcore_axis_name="s", num_cores=_NC
    )

    @functools.partial(
        pl.kernel,
        mesh=mesh,
        compiler_params=pltpu.CompilerParams(use_tc_tiling_on_sc=True),
        out_type=jax.ShapeDtypeStruct((B, DIM), jnp.float32),
        scratch_types=[
            pltpu.VMEM((bpw,), jnp.int32),
            pltpu.VMEM((chunk, PDIM), jnp.float32),
            pltpu.VMEM((chunk, PDIM), jnp.float32),
            pltpu.VMEM((chunk, DIM), jnp.float32),
            pltpu.VMEM((chunk, DIM), jnp.float32),
            pltpu.SemaphoreType.DMA,
            pltpu.SemaphoreType.DMA,
            pltpu.SemaphoreType.DMA,
            pltpu.SemaphoreType.DMA,
        ],
    )
    def emb(x_hbm, table_hbm, out_hbm, idx_v, in0, in1, out0, out1,
            gsem0, gsem1, ssem0, ssem1):
        ins = (in0, in1)
        outs = (out0, out1)
        gsems = (gsem0, gsem1)
        ssems = (ssem0, ssem1)

        wid = lax.axis_index("s") * _NC + lax.axis_index("c")
        base = wid * bpw
        pltpu.sync_copy(x_hbm.at[pl.ds(base, bpw)], idx_v)

        def gather_start(off, buf, sem):
            pltpu.async_copy(table_hbm.at[idx_v.at[pl.ds(off, chunk)]], buf, sem)

        for b in range(2):
            gather_start(pl.multiple_of(b * chunk, chunk), ins[b], gsems[b])

        def body(g0, carry):
            for b in range(2):
                g = g0 * 2 + b
                off = pl.multiple_of(g * chunk, chunk)
                inb, outb = ins[b], outs[b]
                pltpu.make_async_copy(
                    table_hbm.at[idx_v.at[pl.ds(off, chunk)]], inb, gsems[b]
                ).wait()

                @pl.when(g0 > 0)
                def _():
                    pltpu.make_async_copy(
                        outb, out_hbm.at[pl.ds(base, chunk)], ssems[b]
                    ).wait()

                def _scale(i0, c):
                    for u in range(unroll):
                        i = i0 * unroll + u
                        for j in range(DIM // _L):
                            sl = pl.ds(j * _L, _L)
                            outb[i, sl] = inb[i, sl] * SCALE
                    return c

                lax.fori_loop(0, chunk // unroll, _scale, 0)

                pltpu.async_copy(outb, out_hbm.at[pl.ds(base + off, chunk)], ssems[b])

                @pl.when(g + 2 < nch)
                def _():
                    off2 = pl.multiple_of((g + 2) * chunk, chunk)
                    gather_start(off2, inb, gsems[b])

            return carry

        lax.fori_loop(0, nch // 2, body, 0)

        for b in range(2):
            pltpu.make_async_copy(
                outs[b], out_hbm.at[pl.ds(base, chunk)], ssems[b]
            ).wait()

    return emb


@functools.lru_cache(maxsize=None)
def _make_transpose(V: int):
    blk = 32768
    grid = (V + blk - 1) // blk

    def tk(tab_ref, out_ref):
        t = jnp.transpose(tab_ref[...], (1, 0))
        out_ref[...] = jnp.concatenate(
            [t, jnp.zeros((blk, PDIM - DIM), jnp.float32)], axis=1
        )

    return pl.pallas_call(
        tk,
        grid=(grid,),
        in_specs=[pl.BlockSpec((DIM, blk), lambda i: (0, i))],
        out_specs=pl.BlockSpec((blk, PDIM), lambda i: (i, 0)),
        out_shape=jax.ShapeDtypeStruct((V, PDIM), jnp.float32),
    )


def kernel(x, table):
    B = x.size
    V = table.shape[0]
    flat = x.reshape(B).astype(jnp.int32)
    tab2 = _make_transpose(V)(table.T)
    out = _make_kernel(B, V)(flat, tab2)
    return out.reshape(x.shape + (DIM,))
